# baseline (speedup 1.0000x reference)
import functools

import jax
import jax.numpy as jnp
from jax import lax
from jax.experimental import pallas as pl
from jax.experimental.pallas import tpu as pltpu
from jax.experimental.pallas import tpu_sc as plsc

B, C, H, W = 4, 64, 24, 24
HW = H * W
K = 512
_N_ELEM = B * C * HW

_NW = 32
_CPW = C // 8


def _vq_tc_kernel(x_ref, emb_ref, idx_ref, loss_ref):
    b = pl.program_id(0)
    xb = x_ref[0]
    emb = emb_ref[...]

    scores = lax.dot_general(
        emb, xb, (((1,), (0,)), ((), ())),
        preferred_element_type=jnp.float32,
        precision=lax.Precision.HIGHEST,
    )
    en = jnp.sum(emb * emb, axis=1, keepdims=True)
    d = en - 2.0 * scores

    rowids = lax.broadcasted_iota(jnp.int32, (K, HW), 0)
    big = jnp.int32(K)

    dmin1 = jnp.min(d, axis=0, keepdims=True)
    i1 = jnp.min(jnp.where(d == dmin1, rowids, big), axis=0, keepdims=True)
    oh1 = (rowids == i1).astype(jnp.float32)
    e1 = lax.dot_general(
        emb, oh1, (((0,), (0,)), ((), ())),
        preferred_element_type=jnp.float32,
        precision=lax.Precision.HIGHEST,
    )

    dm = jnp.where(rowids == i1, jnp.float32(jnp.inf), d)
    dmin2 = jnp.min(dm, axis=0, keepdims=True)
    i2 = jnp.min(jnp.where(dm == dmin2, rowids, big), axis=0, keepdims=True)
    oh2 = (rowids == i2).astype(jnp.float32)
    e2 = lax.dot_general(
        emb, oh2, (((0,), (0,)), ((), ())),
        preferred_element_type=jnp.float32,
        precision=lax.Precision.HIGHEST,
    )

    d1 = jnp.sum((xb - e1) ** 2, axis=0, keepdims=True)
    d2 = jnp.sum((xb - e2) ** 2, axis=0, keepdims=True)
    win2 = (d2 < d1) | ((d2 == d1) & (i2 < i1))

    idx_ref[0] = jnp.where(win2, i2, i1)

    dwin = jnp.where(win2, d2, d1)
    part = jnp.sum(dwin, axis=1, keepdims=True)

    @pl.when(b == 0)
    def _init():
        loss_ref[...] = jnp.zeros((1, 1), jnp.float32)

    loss_ref[...] += part

    @pl.when(b == pl.num_programs(0) - 1)
    def _fin():
        loss_ref[...] = loss_ref[...] * jnp.float32(2.0 / _N_ELEM)


@functools.partial(
    pl.kernel,
    out_type=jax.ShapeDtypeStruct((B, C, HW), jnp.float32),
    mesh=plsc.VectorSubcoreMesh(core_axis_name="c", subcore_axis_name="s"),
    scratch_types=[
        pltpu.VMEM((K, C), jnp.float32),
        pltpu.VMEM((HW,), jnp.int32),
        pltpu.VMEM((_CPW, HW), jnp.float32),
    ],
    compiler_params=pltpu.CompilerParams(needs_layout_passes=False),
)
def _sc_gather(emb_hbm, idx_hbm, out_hbm, emb_v, idx_v, out_v):
    wid = lax.axis_index("s") * 2 + lax.axis_index("c")
    b = wid // (_NW // B)
    c0 = (wid % (_NW // B)) * _CPW
    pltpu.sync_copy(emb_hbm, emb_v)
    pltpu.sync_copy(idx_hbm.at[b, 0], idx_v)

    for j in range(HW // 16):
        idx16 = idx_v[pl.ds(j * 16, 16)]
        for cc in range(_CPW):
            col = jnp.zeros((16,), jnp.int32) + (c0 + cc)
            out_v[cc, pl.ds(j * 16, 16)] = plsc.load_gather(emb_v, [idx16, col])

    pltpu.sync_copy(out_v, out_hbm.at[b, pl.ds(c0, _CPW), :])


@jax.jit
def kernel(x, embeddings):
    x3 = x.reshape(B, C, HW)
    idx3, loss = pl.pallas_call(
        _vq_tc_kernel,
        grid=(B,),
        in_specs=[
            pl.BlockSpec((1, C, HW), lambda b: (b, 0, 0)),
            pl.BlockSpec((K, C), lambda b: (0, 0)),
        ],
        out_specs=[
            pl.BlockSpec((1, 1, HW), lambda b: (b, 0, 0)),
            pl.BlockSpec((1, 1), lambda b: (0, 0)),
        ],
        out_shape=[
            jax.ShapeDtypeStruct((B, 1, HW), jnp.int32),
            jax.ShapeDtypeStruct((1, 1), jnp.float32),
        ],
    )(x3, embeddings)
    zq3 = _sc_gather(embeddings, idx3)
    return zq3.reshape(B, C, H, W), loss[0, 0]


# speedup vs baseline: 2.4188x; 2.4188x over previous
import functools

import jax
import jax.numpy as jnp
from jax import lax
from jax.experimental import pallas as pl

B, C, H, W = 4, 64, 24, 24
HW = H * W
K = 512
_N_ELEM = B * C * HW


def _vq_tc_kernel(x_ref, emb_ref, zq_ref, loss_ref):
    emb = emb_ref[...]
    en = jnp.sum(emb * emb, axis=1, keepdims=True)
    rowids = lax.broadcasted_iota(jnp.int32, (K, HW), 0)
    big = jnp.int32(K)

    total = jnp.zeros((1, 1), jnp.float32)
    for b in range(B):
        xb = x_ref[b]

        scores = lax.dot_general(
            emb, xb, (((1,), (0,)), ((), ())),
            preferred_element_type=jnp.float32,
            precision=lax.Precision.HIGHEST,
        )
        d = en - 2.0 * scores

        dmin1 = jnp.min(d, axis=0, keepdims=True)
        i1 = jnp.min(jnp.where(d == dmin1, rowids, big), axis=0, keepdims=True)
        oh1 = (rowids == i1).astype(jnp.float32)
        e1 = lax.dot_general(
            emb, oh1, (((0,), (0,)), ((), ())),
            preferred_element_type=jnp.float32,
            precision=lax.Precision.HIGHEST,
        )

        dm = jnp.where(rowids == i1, jnp.float32(jnp.inf), d)
        dmin2 = jnp.min(dm, axis=0, keepdims=True)
        i2 = jnp.min(jnp.where(dm == dmin2, rowids, big), axis=0, keepdims=True)
        oh2 = (rowids == i2).astype(jnp.float32)
        e2 = lax.dot_general(
            emb, oh2, (((0,), (0,)), ((), ())),
            preferred_element_type=jnp.float32,
            precision=lax.Precision.HIGHEST,
        )

        d1 = jnp.sum((xb - e1) ** 2, axis=0, keepdims=True)
        d2 = jnp.sum((xb - e2) ** 2, axis=0, keepdims=True)
        win2 = (d2 < d1) | ((d2 == d1) & (i2 < i1))

        zq_ref[b] = jnp.where(win2, e2, e1)

        dwin = jnp.where(win2, d2, d1)
        total = total + jnp.sum(dwin, axis=1, keepdims=True)

    loss_ref[...] = total * jnp.float32(2.0 / _N_ELEM)


@jax.jit
def kernel(x, embeddings):
    x3 = x.reshape(B, C, HW)
    zq3, loss = pl.pallas_call(
        _vq_tc_kernel,
        out_shape=[
            jax.ShapeDtypeStruct((B, C, HW), jnp.float32),
            jax.ShapeDtypeStruct((1, 1), jnp.float32),
        ],
    )(x3, embeddings)
    return zq3.reshape(B, C, H, W), loss[0, 0]
